# baseline (speedup 1.0000x reference)
import jax
import jax.numpy as jnp
from jax import lax
from jax.experimental import pallas as pl
from jax.experimental.pallas import tpu as pltpu
from jax.experimental.pallas import tpu_sc as plsc

N = 10000
E = 320000
D = 128
H = 256
C = 40

NS = 16
K = 125
NCHUNK = E // K
RPS = (N // NS) // 8 * 8
TAIL = N - NS * RPS
TAIL_BASE = NS * RPS

_MESH = plsc.VectorSubcoreMesh(
    core_axis_name="c", subcore_axis_name="s", num_cores=2, num_subcores=NS)


G = 8


def _sc_scratch(rows_t, acc_t):
  return [
      pltpu.VMEM((2, G, K), jnp.int32),
      pltpu.VMEM((2, G, K), jnp.int32),
      rows_t,
      acc_t,
      pltpu.SemaphoreType.DMA,
      pltpu.SemaphoreType.DMA,
      pltpu.SemaphoreType.DMA,
      pltpu.SemaphoreType.DMA,
  ]


def _run_propagate(tbl, out, src_hbm, dst_hbm, srcb, dstb, rows_v, acc,
                   sem0, sem1, semis, semid, s, chunk_lo, nps):
  base = pl.multiple_of(s * RPS, 8)
  pltpu.sync_copy(tbl.at[pl.ds(base, RPS)], acc.at[pl.ds(base, RPS)])

  @pl.when(s == 0)
  def _():
    pltpu.sync_copy(tbl.at[pl.ds(TAIL_BASE, TAIL)],
                    acc.at[pl.ds(TAIL_BASE, TAIL)])

  nsup = nps // G
  row0 = chunk_lo + s * nps

  def idx_copy(sup, q, hbm, buf, sem):
    r = pl.multiple_of(row0 + sup * G, 8)
    return pltpu.make_async_copy(hbm.at[pl.ds(r, G)], buf.at[q], sem)

  idx_copy(0, 0, src_hbm, srcb, semis).start()
  idx_copy(0, 0, dst_hbm, dstb, semid).start()
  idx_copy(0, 0, src_hbm, srcb, semis).wait()
  idx_copy(0, 0, dst_hbm, dstb, semid).wait()
  idx_copy(1, 1, src_hbm, srcb, semis).start()
  idx_copy(1, 1, dst_hbm, dstb, semid).start()

  plsc.subcore_barrier()

  sems = (sem0, sem1)

  def gather(q, ch, b):
    return pltpu.make_async_copy(tbl.at[srcb.at[q].at[ch]], rows_v.at[b],
                                 sems[b])

  gather(0, 0, 0).start()
  gather(0, 1, 1).start()

  def body(u, carry):
    for su in range(2):
      q = su
      sup = 2 * u + su
      if su == 0:
        @pl.when(u > 0)
        def _():
          idx_copy(sup + 1, 1, src_hbm, srcb, semis).start()
          idx_copy(sup + 1, 1, dst_hbm, dstb, semid).start()
      else:
        @pl.when(sup + 1 < nsup)
        def _():
          idx_copy(sup + 1, 0, src_hbm, srcb, semis).start()
          idx_copy(sup + 1, 0, dst_hbm, dstb, semid).start()

      for ch in range(G):
        b = ch % 2
        gather(q, ch, b).wait()
        pltpu.sync_copy(rows_v.at[b], acc.at[dstb.at[q].at[ch]], add=True)
        if ch == G - 2:
          @pl.when(sup + 1 < nsup)
          def _():
            idx_copy(sup + 1, 1 - q, src_hbm, srcb, semis).wait()
            idx_copy(sup + 1, 1 - q, dst_hbm, dstb, semid).wait()
        if ch < G - 2:
          gather(q, ch + 2, b).start()
        else:
          @pl.when(sup + 1 < nsup)
          def _():
            gather(1 - q, ch + 2 - G, b).start()
    return carry

  lax.fori_loop(0, nsup // 2, body, 0)
  plsc.subcore_barrier()
  pltpu.sync_copy(acc.at[pl.ds(base, RPS)], out.at[pl.ds(base, RPS)])

  @pl.when(s == 0)
  def _():
    pltpu.sync_copy(acc.at[pl.ds(TAIL_BASE, TAIL)],
                    out.at[pl.ds(TAIL_BASE, TAIL)])


def _sc_l1_body(tbl, src_hbm, dst_hbm, out0, out1,
                srcb, dstb, rows_v, acc, sem0, sem1, semis, semid):
  c = lax.axis_index("c")
  s = lax.axis_index("s")
  nps = NCHUNK // 2 // NS

  @pl.when(c == 0)
  def _():
    _run_propagate(tbl, out0, src_hbm, dst_hbm, srcb, dstb, rows_v, acc,
                   sem0, sem1, semis, semid, s, 0, nps)

  @pl.when(c == 1)
  def _():
    _run_propagate(tbl, out1, src_hbm, dst_hbm, srcb, dstb, rows_v, acc,
                   sem0, sem1, semis, semid, s, NCHUNK // 2, nps)


_sc_prop_l1 = pl.kernel(
    _sc_l1_body,
    out_type=[jax.ShapeDtypeStruct((N, D), jnp.float32)] * 2,
    mesh=_MESH,
    scratch_types=_sc_scratch(pltpu.VMEM((2, K, 128), jnp.float32),
                              pltpu.VMEM_SHARED((N, 128), jnp.float32)),
)

def _sc_l2_body(tblA, tblB, src_hbm, dst_hbm, outA, outB,
                srcb, dstb, rows_v, acc, sem0, sem1, semis, semid):
  c = lax.axis_index("c")
  s = lax.axis_index("s")
  nps = NCHUNK // NS

  @pl.when(c == 0)
  def _():
    _run_propagate(tblA, outA, src_hbm, dst_hbm, srcb, dstb, rows_v, acc,
                   sem0, sem1, semis, semid, s, 0, nps)

  @pl.when(c == 1)
  def _():
    _run_propagate(tblB, outB, src_hbm, dst_hbm, srcb, dstb, rows_v, acc,
                   sem0, sem1, semis, semid, s, 0, nps)


_sc_prop_l2 = pl.kernel(
    _sc_l2_body,
    out_type=[jax.ShapeDtypeStruct((N, H // 2), jnp.float32)] * 2,
    mesh=_MESH,
    scratch_types=_sc_scratch(pltpu.VMEM((2, K, 128), jnp.float32),
                              pltpu.VMEM_SHARED((N, 128), jnp.float32)),
)


def _norm_body(x_ref, o_ref):
  x = x_ref[...]
  n = jnp.sqrt(jnp.sum(x * x, axis=1, keepdims=True))
  o_ref[...] = x / jnp.maximum(n, 1e-12)


def _bmm(a, b):
  return jnp.dot(a.astype(jnp.bfloat16), b.astype(jnp.bfloat16),
                 preferred_element_type=jnp.float32)


def _mid_body(p0, p1, hn, x_ref, wl, b1, wr, w2r, b2, hr_ref, h2a, h2b):
  agg = p0[...] + p1[...] - hn[...]
  out1 = _bmm(agg, wl[...]) + _bmm(x_ref[...], wr[...]) + b1[...]
  n1 = jnp.sqrt(jnp.sum(out1 * out1, axis=1, keepdims=True))
  h = jnp.maximum(out1 / jnp.maximum(n1, 1e-12), 0.0)
  n2 = jnp.sqrt(jnp.sum(h * h, axis=1, keepdims=True))
  hn2 = h / jnp.maximum(n2, 1e-12)
  hr_ref[...] = _bmm(h, w2r[...]) + b2[...]
  h2a[...] = hn2[:, : H // 2]
  h2b[...] = hn2[:, H // 2:]


def _final_body(aggA, aggB, hr, wla, wlb, o_ref):
  out2 = _bmm(aggA[...], wla[...]) + _bmm(aggB[...], wlb[...]) + hr[...]
  n = jnp.sqrt(jnp.sum(out2 * out2, axis=1, keepdims=True))
  o_ref[...] = out2 / jnp.maximum(n, 1e-12)


BLK = 1000
_GRID = (N // BLK,)


def _row_spec(cols):
  return pl.BlockSpec((BLK, cols), lambda i: (i, 0))


def _full_spec(r, c):
  return pl.BlockSpec((r, c), lambda i: (0, 0))


_tc_norm = pl.pallas_call(
    _norm_body,
    grid=_GRID,
    in_specs=[_row_spec(D)],
    out_specs=[_row_spec(D)],
    out_shape=[jax.ShapeDtypeStruct((N, D), jnp.float32)],
)

_tc_mid = pl.pallas_call(
    _mid_body,
    grid=_GRID,
    in_specs=[_row_spec(D), _row_spec(D), _row_spec(D), _row_spec(D),
              _full_spec(D, H), _full_spec(1, H), _full_spec(D, H),
              _full_spec(H, C), _full_spec(1, C)],
    out_specs=[_row_spec(C), _row_spec(H // 2), _row_spec(H // 2)],
    out_shape=[jax.ShapeDtypeStruct((N, C), jnp.float32),
               jax.ShapeDtypeStruct((N, H // 2), jnp.float32),
               jax.ShapeDtypeStruct((N, H // 2), jnp.float32)],
)

_tc_final = pl.pallas_call(
    _final_body,
    grid=_GRID,
    in_specs=[_row_spec(H // 2), _row_spec(H // 2), _row_spec(C),
              _full_spec(H // 2, C), _full_spec(H // 2, C)],
    out_specs=[_row_spec(C)],
    out_shape=[jax.ShapeDtypeStruct((N, C), jnp.float32)],
)


@jax.jit
def kernel(x, edge_index, W1l, b1, W1r, W2l, b2, W2r):
  src = edge_index[0].reshape(NCHUNK, K)
  dst = edge_index[1].reshape(NCHUNK, K)

  (hn,) = _tc_norm(x)
  p0, p1 = _sc_prop_l1(hn, src, dst)
  hr, hn2A, hn2B = _tc_mid(p0, p1, hn, x, W1l, b1.reshape(1, H), W1r,
                           W2r, b2.reshape(1, C))

  agg2A, agg2B = _sc_prop_l2(hn2A, hn2B, src, dst)

  (out,) = _tc_final(agg2A, agg2B, hr, W2l[: H // 2], W2l[H // 2:])
  return out


# speedup vs baseline: 1.0163x; 1.0163x over previous
import jax
import jax.numpy as jnp
from jax import lax
from jax.experimental import pallas as pl
from jax.experimental.pallas import tpu as pltpu
from jax.experimental.pallas import tpu_sc as plsc

N = 10000
E = 320000
D = 128
H = 256
C = 40

NS = 16
K = 125
NCHUNK = E // K
RPS = (N // NS) // 8 * 8
TAIL = N - NS * RPS
TAIL_BASE = NS * RPS

_MESH = plsc.VectorSubcoreMesh(
    core_axis_name="c", subcore_axis_name="s", num_cores=2, num_subcores=NS)


G = 8


def _sc_scratch(rows_t, acc_t):
  return [
      pltpu.VMEM((2, G, K), jnp.int32),
      pltpu.VMEM((2, G, K), jnp.int32),
      rows_t,
      acc_t,
      pltpu.SemaphoreType.DMA,
      pltpu.SemaphoreType.DMA,
      pltpu.SemaphoreType.DMA,
      pltpu.SemaphoreType.DMA,
  ]


def _run_propagate(tbl, out, src_hbm, dst_hbm, srcb, dstb, rows_v, acc,
                   sem0, sem1, semis, semid, s, chunk_lo, nps):
  base = pl.multiple_of(s * RPS, 8)
  pltpu.sync_copy(tbl.at[pl.ds(base, RPS)], acc.at[pl.ds(base, RPS)])

  @pl.when(s == 0)
  def _():
    pltpu.sync_copy(tbl.at[pl.ds(TAIL_BASE, TAIL)],
                    acc.at[pl.ds(TAIL_BASE, TAIL)])

  nsup = nps // G
  row0 = chunk_lo + s * nps

  def idx_copy(sup, q, hbm, buf, sem):
    r = pl.multiple_of(row0 + sup * G, 8)
    return pltpu.make_async_copy(hbm.at[pl.ds(r, G)], buf.at[q], sem)

  idx_copy(0, 0, src_hbm, srcb, semis).start()
  idx_copy(0, 0, dst_hbm, dstb, semid).start()
  idx_copy(0, 0, src_hbm, srcb, semis).wait()
  idx_copy(0, 0, dst_hbm, dstb, semid).wait()
  idx_copy(1, 1, src_hbm, srcb, semis).start()
  idx_copy(1, 1, dst_hbm, dstb, semid).start()

  plsc.subcore_barrier()

  sems = (sem0, sem1)

  def gather(q, ch, b):
    return pltpu.make_async_copy(tbl.at[srcb.at[q].at[ch]], rows_v.at[b],
                                 sems[b])

  gather(0, 0, 0).start()
  gather(0, 1, 1).start()

  def body(u, carry):
    for su in range(2):
      q = su
      sup = 2 * u + su
      if su == 0:
        @pl.when(u > 0)
        def _():
          idx_copy(sup + 1, 1, src_hbm, srcb, semis).start()
          idx_copy(sup + 1, 1, dst_hbm, dstb, semid).start()
      else:
        @pl.when(sup + 1 < nsup)
        def _():
          idx_copy(sup + 1, 0, src_hbm, srcb, semis).start()
          idx_copy(sup + 1, 0, dst_hbm, dstb, semid).start()

      for ch in range(G):
        b = ch % 2
        gather(q, ch, b).wait()
        pltpu.sync_copy(rows_v.at[b], acc.at[dstb.at[q].at[ch]], add=True)
        if ch == G - 2:
          @pl.when(sup + 1 < nsup)
          def _():
            idx_copy(sup + 1, 1 - q, src_hbm, srcb, semis).wait()
            idx_copy(sup + 1, 1 - q, dst_hbm, dstb, semid).wait()
        if ch < G - 2:
          gather(q, ch + 2, b).start()
        else:
          @pl.when(sup + 1 < nsup)
          def _():
            gather(1 - q, ch + 2 - G, b).start()
    return carry

  lax.fori_loop(0, nsup // 2, body, 0)
  plsc.subcore_barrier()
  pltpu.sync_copy(acc.at[pl.ds(base, RPS)], out.at[pl.ds(base, RPS)])

  @pl.when(s == 0)
  def _():
    pltpu.sync_copy(acc.at[pl.ds(TAIL_BASE, TAIL)],
                    out.at[pl.ds(TAIL_BASE, TAIL)])


def _sc_l1_body(tbl, src_hbm, dst_hbm, out0, out1,
                srcb, dstb, rows_v, acc, sem0, sem1, semis, semid):
  c = lax.axis_index("c")
  s = lax.axis_index("s")
  nps = NCHUNK // 2 // NS

  @pl.when(c == 0)
  def _():
    _run_propagate(tbl, out0, src_hbm, dst_hbm, srcb, dstb, rows_v, acc,
                   sem0, sem1, semis, semid, s, 0, nps)

  @pl.when(c == 1)
  def _():
    _run_propagate(tbl, out1, src_hbm, dst_hbm, srcb, dstb, rows_v, acc,
                   sem0, sem1, semis, semid, s, NCHUNK // 2, nps)


_sc_prop_l1 = pl.kernel(
    _sc_l1_body,
    out_type=[jax.ShapeDtypeStruct((N, D), jnp.float32)] * 2,
    mesh=_MESH,
    scratch_types=_sc_scratch(pltpu.VMEM((2, K, 128), jnp.float32),
                              pltpu.VMEM_SHARED((N, 128), jnp.float32)),
)

def _sc_l2_body(tblA, tblB, src_hbm, dst_hbm, outA, outB,
                srcb, dstb, rows_v, acc, sem0, sem1, semis, semid):
  c = lax.axis_index("c")
  s = lax.axis_index("s")
  nps = NCHUNK // NS

  @pl.when(c == 0)
  def _():
    _run_propagate(tblA, outA, src_hbm, dst_hbm, srcb, dstb, rows_v, acc,
                   sem0, sem1, semis, semid, s, 0, nps)

  @pl.when(c == 1)
  def _():
    _run_propagate(tblB, outB, src_hbm, dst_hbm, srcb, dstb, rows_v, acc,
                   sem0, sem1, semis, semid, s, 0, nps)


_sc_prop_l2 = pl.kernel(
    _sc_l2_body,
    out_type=[jax.ShapeDtypeStruct((N, H // 2), jnp.float32)] * 2,
    mesh=_MESH,
    scratch_types=_sc_scratch(pltpu.VMEM((2, K, 128), jnp.float32),
                              pltpu.VMEM_SHARED((N, 128), jnp.float32)),
)


def _norm_body(x_ref, o_ref):
  x = x_ref[...]
  n = jnp.sqrt(jnp.sum(x * x, axis=1, keepdims=True))
  o_ref[...] = x / jnp.maximum(n, 1e-12)


def _bmm(a, b):
  return jnp.dot(a.astype(jnp.bfloat16), b.astype(jnp.bfloat16),
                 preferred_element_type=jnp.float32)


def _mid_body(p0, p1, hn, x_ref, wl, b1, wr, w2r, b2, hr_ref, h2a, h2b):
  agg = p0[...] + p1[...] - hn[...]
  out1 = _bmm(agg, wl[...]) + _bmm(x_ref[...], wr[...]) + b1[...]
  n1 = jnp.sqrt(jnp.sum(out1 * out1, axis=1, keepdims=True))
  h = jnp.maximum(out1 / jnp.maximum(n1, 1e-12), 0.0)
  n2 = jnp.sqrt(jnp.sum(h * h, axis=1, keepdims=True))
  hn2 = h / jnp.maximum(n2, 1e-12)
  hr_ref[...] = _bmm(h, w2r[...]) + b2[...]
  h2a[...] = hn2[:, : H // 2]
  h2b[...] = hn2[:, H // 2:]


def _final_body(aggA, aggB, hr, wla, wlb, o_ref):
  out2 = _bmm(aggA[...], wla[...]) + _bmm(aggB[...], wlb[...]) + hr[...]
  n = jnp.sqrt(jnp.sum(out2 * out2, axis=1, keepdims=True))
  o_ref[...] = out2 / jnp.maximum(n, 1e-12)


BLK = 2000
_GRID = (N // BLK,)


def _row_spec(cols):
  return pl.BlockSpec((BLK, cols), lambda i: (i, 0))


def _full_spec(r, c):
  return pl.BlockSpec((r, c), lambda i: (0, 0))


_tc_norm = pl.pallas_call(
    _norm_body,
    grid=_GRID,
    in_specs=[_row_spec(D)],
    out_specs=[_row_spec(D)],
    out_shape=[jax.ShapeDtypeStruct((N, D), jnp.float32)],
)

_tc_mid = pl.pallas_call(
    _mid_body,
    grid=_GRID,
    in_specs=[_row_spec(D), _row_spec(D), _row_spec(D), _row_spec(D),
              _full_spec(D, H), _full_spec(1, H), _full_spec(D, H),
              _full_spec(H, C), _full_spec(1, C)],
    out_specs=[_row_spec(C), _row_spec(H // 2), _row_spec(H // 2)],
    out_shape=[jax.ShapeDtypeStruct((N, C), jnp.float32),
               jax.ShapeDtypeStruct((N, H // 2), jnp.float32),
               jax.ShapeDtypeStruct((N, H // 2), jnp.float32)],
)

_tc_final = pl.pallas_call(
    _final_body,
    grid=_GRID,
    in_specs=[_row_spec(H // 2), _row_spec(H // 2), _row_spec(C),
              _full_spec(H // 2, C), _full_spec(H // 2, C)],
    out_specs=[_row_spec(C)],
    out_shape=[jax.ShapeDtypeStruct((N, C), jnp.float32)],
)


@jax.jit
def kernel(x, edge_index, W1l, b1, W1r, W2l, b2, W2r):
  src = edge_index[0].reshape(NCHUNK, K)
  dst = edge_index[1].reshape(NCHUNK, K)

  (hn,) = _tc_norm(x)
  p0, p1 = _sc_prop_l1(hn, src, dst)
  hr, hn2A, hn2B = _tc_mid(p0, p1, hn, x, W1l, b1.reshape(1, H), W1r,
                           W2r, b2.reshape(1, C))

  agg2A, agg2B = _sc_prop_l2(hn2A, hn2B, src, dst)

  (out,) = _tc_final(agg2A, agg2B, hr, W2l[: H // 2], W2l[H // 2:])
  return out
